# Initial kernel scaffold; baseline (speedup 1.0000x reference)
import jax
import jax.numpy as jnp
from jax.experimental import pallas as pl


def kernel(x, edge_index, edge_weight, batch, W1, b1, W2, b2, Wout, bout):
    raise NotImplementedError("write your pallas kernel here")


# speedup vs baseline: 11.5348x; 11.5348x over previous
import functools

import jax
import jax.numpy as jnp
from jax import lax
from jax.experimental import pallas as pl
from jax.experimental.pallas import tpu as pltpu
from jax.experimental.pallas import tpu_sc as plsc

N, E, D, H, O, G = 10000, 320000, 128, 128, 64, 16
NC, NS = 2, 16
NW = NC * NS
EPW = E // NW
K = 80
NCHUNK = EPW // K
NPAD = 10240
RPT = NPAD // NS
ZR = 128
F32 = jnp.float32
I32 = jnp.int32

def _deg_body(dst_hbm, w_hbm, out_hbm, dstv, wv, stage, deg_sh):
    c = lax.axis_index("c")
    s = lax.axis_index("s")
    wid = c * NS + s
    base = wid * EPW

    @pl.when(s == 0)
    def _zero():
        def zrow(i, _):
            stage[pl.ds(i * 16, 16)] = jnp.zeros((16,), F32)
            return 0
        lax.fori_loop(0, N // 16, zrow, 0)
        pltpu.sync_copy(stage, deg_sh)

    plsc.subcore_barrier()

    pltpu.sync_copy(w_hbm.at[pl.ds(base, EPW)], wv)

    def chunk(i, _):
        pltpu.sync_copy(dst_hbm.at[pl.ds(base + i * K, K)], dstv.at[0])
        pltpu.sync_copy(wv.at[pl.ds(i * K, K)], deg_sh.at[dstv.at[0]], add=True)
        return 0
    lax.fori_loop(0, NCHUNK, chunk, 0)

    plsc.subcore_barrier()

    @pl.when(s == 0)
    def _writeout():
        pltpu.sync_copy(deg_sh, stage)
        pltpu.sync_copy(stage, out_hbm.at[c])


def _edge_body(hp_hbm, src_hbm, dst_hbm, w_hbm, out_hbm,
               srcv, dstv, wv, rows, zbuf, acc_sh, sem):
    c = lax.axis_index("c")
    s = lax.axis_index("s")
    wid = c * NS + s
    base = wid * EPW
    row0 = s * RPT

    def zrow(i, _):
        for j in range(8):
            zbuf[i, pl.ds(j * 16, 16)] = jnp.zeros((16,), F32)
        return 0
    lax.fori_loop(0, ZR, zrow, 0)
    for q in range(RPT // ZR):
        pltpu.sync_copy(zbuf, acc_sh.at[pl.ds(row0 + q * ZR, ZR)])

    plsc.subcore_barrier()

    pltpu.sync_copy(src_hbm.at[pl.ds(base, EPW)], srcv)
    pltpu.sync_copy(w_hbm.at[pl.ds(base, EPW)], wv)

    def chunk(i, _):
        pltpu.sync_copy(dst_hbm.at[pl.ds(base + i * K, K)], dstv.at[0])
        pltpu.async_copy(hp_hbm.at[srcv.at[pl.ds(i * K, K)]], rows, sem).wait()

        def scale(e, _):
            wsplat = plsc.load_gather(wv, [jnp.full((16,), i * K + e, I32)])
            for j in range(8):
                sl = pl.ds(j * 16, 16)
                rows[e, sl] = rows[e, sl] * wsplat
            return 0
        lax.fori_loop(0, K, scale, 0)

        pltpu.sync_copy(rows, acc_sh.at[dstv.at[0]], add=True)
        return 0
    lax.fori_loop(0, NCHUNK, chunk, 0)

    plsc.subcore_barrier()

    for q in range(RPT // ZR):
        pltpu.sync_copy(acc_sh.at[pl.ds(row0 + q * ZR, ZR)], zbuf)
        pltpu.sync_copy(zbuf, out_hbm.at[c, pl.ds(row0 + q * ZR, ZR)])


@functools.cache
def _sc_kernels():
    mesh = plsc.VectorSubcoreMesh(
        core_axis_name="c", subcore_axis_name="s",
        num_cores=NC, num_subcores=NS)
    deg = pl.kernel(
        _deg_body,
        out_type=jax.ShapeDtypeStruct((NC, N), F32),
        mesh=mesh,
        scratch_types=[
            pltpu.VMEM((1, K), I32),
            pltpu.VMEM((EPW,), F32),
            pltpu.VMEM((N,), F32),
            pltpu.VMEM_SHARED((N,), F32),
        ],
        compiler_params=pltpu.CompilerParams(needs_layout_passes=False),
    )
    edge = pl.kernel(
        _edge_body,
        out_type=jax.ShapeDtypeStruct((NC, NPAD, H), F32),
        mesh=mesh,
        scratch_types=[
            pltpu.VMEM((EPW,), I32),
            pltpu.VMEM((1, K), I32),
            pltpu.VMEM((EPW,), F32),
            pltpu.VMEM((K, H), F32),
            pltpu.VMEM((ZR, H), F32),
            pltpu.VMEM_SHARED((NPAD, H), F32),
            pltpu.SemaphoreType.DMA,
        ],
        compiler_params=pltpu.CompilerParams(needs_layout_passes=False),
    )
    return deg, edge


_RB = 1000
_GRID = N // _RB


def _tc_first_body(degT_ref, x_ref, w1_ref, hp_ref, dinv_ref):
    d = degT_ref[...]
    dv = lax.rsqrt(d[:, 0:1] + d[:, 1:2] + 1.0)
    h = jnp.dot(x_ref[...], w1_ref[...], preferred_element_type=F32)
    hp_ref[...] = h * dv
    dinv_ref[...] = dv


def _tc_first(degT, x, W1):
    return pl.pallas_call(
        _tc_first_body,
        grid=(_GRID,),
        in_specs=[
            pl.BlockSpec((_RB, 2), lambda i: (i, 0)),
            pl.BlockSpec((_RB, D), lambda i: (i, 0)),
            pl.BlockSpec((D, H), lambda i: (0, 0)),
        ],
        out_specs=[
            pl.BlockSpec((_RB, H), lambda i: (i, 0)),
            pl.BlockSpec((_RB, 1), lambda i: (i, 0)),
        ],
        out_shape=[
            jax.ShapeDtypeStruct((NPAD, H), F32),
            jax.ShapeDtypeStruct((N, 1), F32),
        ],
    )(degT, x, W1)


def _tc_mid_body(a0_ref, a1_ref, hp_ref, dinv_ref, b1_ref, w2_ref, out_ref):
    dv = dinv_ref[...]
    z = dv * (a0_ref[...] + a1_ref[...] + hp_ref[...]) + b1_ref[...]
    a = jnp.maximum(z, 0.0)
    out_ref[...] = dv * jnp.dot(a, w2_ref[...], preferred_element_type=F32)


def _tc_mid(a0, a1, hp, dinv, b1, W2):
    return pl.pallas_call(
        _tc_mid_body,
        grid=(_GRID,),
        in_specs=[
            pl.BlockSpec((_RB, H), lambda i: (i, 0)),
            pl.BlockSpec((_RB, H), lambda i: (i, 0)),
            pl.BlockSpec((_RB, H), lambda i: (i, 0)),
            pl.BlockSpec((_RB, 1), lambda i: (i, 0)),
            pl.BlockSpec((1, H), lambda i: (0, 0)),
            pl.BlockSpec((H, H), lambda i: (0, 0)),
        ],
        out_specs=pl.BlockSpec((_RB, H), lambda i: (i, 0)),
        out_shape=jax.ShapeDtypeStruct((NPAD, H), F32),
    )(a0, a1, hp, dinv, b1, W2)


def _tc_last_body(a0_ref, a1_ref, hp_ref, dinv_ref, b2_ref, batch_ref,
                  wout_ref, bout_ref, out_ref, sums, cnt):
    i = pl.program_id(0)
    dv = dinv_ref[...]
    z = dv * (a0_ref[...] + a1_ref[...] + hp_ref[...]) + b2_ref[...]
    a = jnp.maximum(z, 0.0)
    brow = batch_ref[0]
    oh = (lax.broadcasted_iota(I32, (G, _RB), 0) == brow).astype(F32)

    @pl.when(i == 0)
    def _init():
        sums[...] = jnp.zeros((G, H), F32)
        cnt[...] = jnp.zeros((G, 1), F32)

    sums[...] = sums[...] + jnp.dot(oh, a, preferred_element_type=F32)
    cnt[...] = cnt[...] + jnp.sum(oh, axis=1, keepdims=True)

    @pl.when(i == _GRID - 1)
    def _final():
        pooled = sums[...] / jnp.maximum(cnt[...], 1.0)
        out_ref[...] = (jnp.dot(pooled, wout_ref[...],
                                preferred_element_type=F32) + bout_ref[...])


def _tc_last(a0, a1, hp, dinv, b2, batch_row, Wout, bout):
    return pl.pallas_call(
        _tc_last_body,
        grid=(_GRID,),
        in_specs=[
            pl.BlockSpec((_RB, H), lambda i: (i, 0)),
            pl.BlockSpec((_RB, H), lambda i: (i, 0)),
            pl.BlockSpec((_RB, H), lambda i: (i, 0)),
            pl.BlockSpec((_RB, 1), lambda i: (i, 0)),
            pl.BlockSpec((1, H), lambda i: (0, 0)),
            pl.BlockSpec((1, 1, _RB), lambda i: (i, 0, 0)),
            pl.BlockSpec((H, O), lambda i: (0, 0)),
            pl.BlockSpec((1, O), lambda i: (0, 0)),
        ],
        out_specs=pl.BlockSpec((G, O), lambda i: (0, 0)),
        out_shape=jax.ShapeDtypeStruct((G, O), F32),
        scratch_shapes=[pltpu.VMEM((G, H), F32), pltpu.VMEM((G, 1), F32)],
        compiler_params=pltpu.CompilerParams(
            dimension_semantics=("arbitrary",)),
    )(a0, a1, hp, dinv, b2, batch_row, Wout, bout)


def kernel(x, edge_index, edge_weight, batch, W1, b1, W2, b2, Wout, bout):
    deg_kernel, edge_kernel = _sc_kernels()
    src = edge_index[0]
    dst = edge_index[1]

    deg_p = deg_kernel(dst, edge_weight)
    degT = deg_p.T

    hp1, dinv = _tc_first(degT, x, W1)
    acc1 = edge_kernel(hp1, src, dst, edge_weight)

    hp2 = _tc_mid(acc1[0], acc1[1], hp1, dinv, b1.reshape(1, H), W2)
    acc2 = edge_kernel(hp2, src, dst, edge_weight)

    return _tc_last(acc2[0], acc2[1], hp2, dinv, b2.reshape(1, H),
                    batch.reshape(_GRID, 1, _RB), Wout, bout.reshape(1, O))


# speedup vs baseline: 20.5907x; 1.7851x over previous
import functools

import jax
import jax.numpy as jnp
from jax import lax
from jax.experimental import pallas as pl
from jax.experimental.pallas import tpu as pltpu
from jax.experimental.pallas import tpu_sc as plsc

N, E, D, H, O, G = 10000, 320000, 128, 128, 64, 16
NC, NS = 2, 16
NW = NC * NS
EPW = E // NW
K = 50
NCHUNK = EPW // K
NBUF = 4
NPAD = 10240
RPT = NPAD // NS
WR = 40
F32 = jnp.float32
I32 = jnp.int32

def _deg_body(dst_hbm, w_hbm, out_hbm, dstv, wv, stage, deg_sh, sem):
    c = lax.axis_index("c")
    s = lax.axis_index("s")
    wid = c * NS + s

    @pl.when(s == 0)
    def _zero():
        def zrow(i, _):
            stage[pl.ds(i * 16, 16)] = jnp.zeros((16,), F32)
            return 0
        lax.fori_loop(0, N // 16, zrow, 0)
        pltpu.sync_copy(stage, deg_sh)

    plsc.subcore_barrier()

    pltpu.sync_copy(dst_hbm.at[wid], dstv)
    pltpu.sync_copy(w_hbm.at[wid], wv)

    def fire(i, _):
        pltpu.async_copy(wv.at[i], deg_sh.at[dstv.at[i]], sem, add=True)
        return 0
    lax.fori_loop(0, NCHUNK, fire, 0)

    def drain(i, _):
        pltpu.make_async_copy(wv.at[0], deg_sh.at[dstv.at[0]], sem).wait()
        return 0
    lax.fori_loop(0, NCHUNK, drain, 0)

    plsc.subcore_barrier()

    @pl.when(s == 0)
    def _writeout():
        pltpu.sync_copy(deg_sh, stage)
        pltpu.sync_copy(stage, out_hbm.at[c])


def _edge_body(hp_hbm, src_hbm, dst_hbm, w_hbm, out_hbm,
               srcv, dstv, wring, r0, r1, r2, r3, acc_sh,
               g0, g1, g2, g3, s0, s1, s2, s3):
    c = lax.axis_index("c")
    s = lax.axis_index("s")
    wid = c * NS + s
    row0 = s * RPT
    rows = (r0, r1, r2, r3)
    gsem = (g0, g1, g2, g3)
    ssem = (s0, s1, s2, s3)

    def zrow(i, _):
        for j in range(8):
            rows[0][i, pl.ds(j * 16, 16)] = jnp.zeros((16,), F32)
        return 0
    lax.fori_loop(0, WR, zrow, 0)

    def zcopy(q, _):
        pltpu.sync_copy(rows[0].at[pl.ds(0, WR)],
                        acc_sh.at[pl.ds(row0 + q * WR, WR)])
        return 0
    lax.fori_loop(0, RPT // WR, zcopy, 0)

    pltpu.sync_copy(src_hbm.at[wid], srcv)
    pltpu.sync_copy(dst_hbm.at[wid], dstv)

    for b in range(2):
        pltpu.async_copy(hp_hbm.at[srcv.at[b]], rows[b], gsem[b])
        pltpu.async_copy(w_hbm.at[wid, b], wring.at[b], gsem[b])

    plsc.subcore_barrier()

    def quad(p, _):
        for b in range(NBUF):
            ch = NBUF * p + b
            rb, b2 = rows[b], (b + 2) % NBUF
            pltpu.make_async_copy(hp_hbm.at[srcv.at[ch]], rb, gsem[b]).wait()
            pltpu.make_async_copy(
                w_hbm.at[wid, 0], wring.at[b], gsem[b]).wait()

            def scale(e, _, _rb=rb, _b=b):
                wsplat = plsc.load_gather(
                    wring, [jnp.full((16,), _b, I32),
                            jnp.full((16,), 0, I32),
                            jnp.full((16,), e, I32)])
                for j in range(8):
                    sl = pl.ds(j * 16, 16)
                    _rb[e, sl] = _rb[e, sl] * wsplat
                return 0
            lax.fori_loop(0, K, scale, 0)

            pltpu.async_copy(rb, acc_sh.at[dstv.at[ch]], ssem[b], add=True)

            @pl.when(ch >= 2)
            def _(_b2=b2):
                pltpu.make_async_copy(
                    rows[_b2], acc_sh.at[dstv.at[0]], ssem[_b2]).wait()

            @pl.when(ch + 2 < NCHUNK)
            def _(_b2=b2, _ch=ch):
                pltpu.async_copy(
                    hp_hbm.at[srcv.at[_ch + 2]], rows[_b2], gsem[_b2])
                pltpu.async_copy(
                    w_hbm.at[wid, _ch + 2], wring.at[_b2], gsem[_b2])
        return 0
    lax.fori_loop(0, NCHUNK // NBUF, quad, 0)

    for b in ((NCHUNK - 2) % NBUF, (NCHUNK - 1) % NBUF):
        pltpu.make_async_copy(rows[b], acc_sh.at[dstv.at[0]], ssem[b]).wait()

    plsc.subcore_barrier()

    def wcopy(q, _):
        pltpu.sync_copy(acc_sh.at[pl.ds(row0 + q * WR, WR)],
                        rows[0].at[pl.ds(0, WR)])
        pltpu.sync_copy(rows[0].at[pl.ds(0, WR)],
                        out_hbm.at[c, pl.ds(row0 + q * WR, WR)])
        return 0
    lax.fori_loop(0, RPT // WR, wcopy, 0)


@functools.cache
def _sc_kernels():
    mesh = plsc.VectorSubcoreMesh(
        core_axis_name="c", subcore_axis_name="s",
        num_cores=NC, num_subcores=NS)
    deg = pl.kernel(
        _deg_body,
        out_type=jax.ShapeDtypeStruct((NC, N), F32),
        mesh=mesh,
        scratch_types=[
            pltpu.VMEM((NCHUNK, K), I32),
            pltpu.VMEM((NCHUNK, K), F32),
            pltpu.VMEM((N,), F32),
            pltpu.VMEM_SHARED((N,), F32),
            pltpu.SemaphoreType.DMA,
        ],
        compiler_params=pltpu.CompilerParams(needs_layout_passes=False),
    )
    edge = pl.kernel(
        _edge_body,
        out_type=jax.ShapeDtypeStruct((NC, NPAD, H), F32),
        mesh=mesh,
        scratch_types=(
            [
                pltpu.VMEM((NCHUNK, K), I32),
                pltpu.VMEM((NCHUNK, K), I32),
                pltpu.VMEM((NBUF, 1, K), F32),
            ]
            + [pltpu.VMEM((K, H), F32)] * NBUF
            + [pltpu.VMEM_SHARED((NPAD, H), F32)]
            + [pltpu.SemaphoreType.DMA] * (2 * NBUF)
        ),
        compiler_params=pltpu.CompilerParams(
            needs_layout_passes=False, use_tc_tiling_on_sc=False),
    )
    return deg, edge


_RB = 1000
_GRID = N // _RB


def _tc_first_body(degT_ref, x_ref, w1_ref, hp_ref, dinv_ref):
    d = degT_ref[...]
    dv = lax.rsqrt(d[:, 0:1] + d[:, 1:2] + 1.0)
    h = jnp.dot(x_ref[...], w1_ref[...], preferred_element_type=F32)
    hp_ref[...] = h * dv
    dinv_ref[...] = dv


def _tc_first(degT, x, W1):
    return pl.pallas_call(
        _tc_first_body,
        grid=(_GRID,),
        in_specs=[
            pl.BlockSpec((_RB, 2), lambda i: (i, 0)),
            pl.BlockSpec((_RB, D), lambda i: (i, 0)),
            pl.BlockSpec((D, H), lambda i: (0, 0)),
        ],
        out_specs=[
            pl.BlockSpec((_RB, H), lambda i: (i, 0)),
            pl.BlockSpec((_RB, 1), lambda i: (i, 0)),
        ],
        out_shape=[
            jax.ShapeDtypeStruct((NPAD, H), F32),
            jax.ShapeDtypeStruct((N, 1), F32),
        ],
    )(degT, x, W1)


def _tc_mid_body(a0_ref, a1_ref, hp_ref, dinv_ref, b1_ref, w2_ref, out_ref):
    dv = dinv_ref[...]
    z = dv * (a0_ref[...] + a1_ref[...] + hp_ref[...]) + b1_ref[...]
    a = jnp.maximum(z, 0.0)
    out_ref[...] = dv * jnp.dot(a, w2_ref[...], preferred_element_type=F32)


def _tc_mid(a0, a1, hp, dinv, b1, W2):
    return pl.pallas_call(
        _tc_mid_body,
        grid=(_GRID,),
        in_specs=[
            pl.BlockSpec((_RB, H), lambda i: (i, 0)),
            pl.BlockSpec((_RB, H), lambda i: (i, 0)),
            pl.BlockSpec((_RB, H), lambda i: (i, 0)),
            pl.BlockSpec((_RB, 1), lambda i: (i, 0)),
            pl.BlockSpec((1, H), lambda i: (0, 0)),
            pl.BlockSpec((H, H), lambda i: (0, 0)),
        ],
        out_specs=pl.BlockSpec((_RB, H), lambda i: (i, 0)),
        out_shape=jax.ShapeDtypeStruct((NPAD, H), F32),
    )(a0, a1, hp, dinv, b1, W2)


def _tc_last_body(a0_ref, a1_ref, hp_ref, dinv_ref, b2_ref, batch_ref,
                  wout_ref, bout_ref, out_ref, sums, cnt):
    i = pl.program_id(0)
    dv = dinv_ref[...]
    z = dv * (a0_ref[...] + a1_ref[...] + hp_ref[...]) + b2_ref[...]
    a = jnp.maximum(z, 0.0)
    brow = batch_ref[0]
    oh = (lax.broadcasted_iota(I32, (G, _RB), 0) == brow).astype(F32)

    @pl.when(i == 0)
    def _init():
        sums[...] = jnp.zeros((G, H), F32)
        cnt[...] = jnp.zeros((G, 1), F32)

    sums[...] = sums[...] + jnp.dot(oh, a, preferred_element_type=F32)
    cnt[...] = cnt[...] + jnp.sum(oh, axis=1, keepdims=True)

    @pl.when(i == _GRID - 1)
    def _final():
        pooled = sums[...] / jnp.maximum(cnt[...], 1.0)
        out_ref[...] = (jnp.dot(pooled, wout_ref[...],
                                preferred_element_type=F32) + bout_ref[...])


def _tc_last(a0, a1, hp, dinv, b2, batch_row, Wout, bout):
    return pl.pallas_call(
        _tc_last_body,
        grid=(_GRID,),
        in_specs=[
            pl.BlockSpec((_RB, H), lambda i: (i, 0)),
            pl.BlockSpec((_RB, H), lambda i: (i, 0)),
            pl.BlockSpec((_RB, H), lambda i: (i, 0)),
            pl.BlockSpec((_RB, 1), lambda i: (i, 0)),
            pl.BlockSpec((1, H), lambda i: (0, 0)),
            pl.BlockSpec((1, 1, _RB), lambda i: (i, 0, 0)),
            pl.BlockSpec((H, O), lambda i: (0, 0)),
            pl.BlockSpec((1, O), lambda i: (0, 0)),
        ],
        out_specs=pl.BlockSpec((G, O), lambda i: (0, 0)),
        out_shape=jax.ShapeDtypeStruct((G, O), F32),
        scratch_shapes=[pltpu.VMEM((G, H), F32), pltpu.VMEM((G, 1), F32)],
        compiler_params=pltpu.CompilerParams(
            dimension_semantics=("arbitrary",)),
    )(a0, a1, hp, dinv, b2, batch_row, Wout, bout)


def kernel(x, edge_index, edge_weight, batch, W1, b1, W2, b2, Wout, bout):
    deg_kernel, edge_kernel = _sc_kernels()
    src3 = edge_index[0].reshape(NW, NCHUNK, K)
    dst3 = edge_index[1].reshape(NW, NCHUNK, K)
    w3 = edge_weight.reshape(NW, NCHUNK, K)
    w4 = edge_weight.reshape(NW, NCHUNK, 1, K)

    deg_p = deg_kernel(dst3, w3)
    degT = deg_p.T

    hp1, dinv = _tc_first(degT, x, W1)
    acc1 = edge_kernel(hp1, src3, dst3, w4)

    hp2 = _tc_mid(acc1[0], acc1[1], hp1, dinv, b1.reshape(1, H), W2)
    acc2 = edge_kernel(hp2, src3, dst3, w4)

    return _tc_last(acc2[0], acc2[1], hp2, dinv, b2.reshape(1, H),
                    batch.reshape(_GRID, 1, _RB), Wout, bout.reshape(1, O))


# speedup vs baseline: 20.9485x; 1.0174x over previous
import functools

import jax
import jax.numpy as jnp
from jax import lax
from jax.experimental import pallas as pl
from jax.experimental.pallas import tpu as pltpu
from jax.experimental.pallas import tpu_sc as plsc

N, E, D, H, O, G = 10000, 320000, 128, 128, 64, 16
NC, NS = 2, 16
NW = NC * NS
EPW = E // NW
K = 50
NCHUNK = EPW // K
NBUF = 4
NPAD = 10240
RPT = NPAD // NS
WR = 40
F32 = jnp.float32
I32 = jnp.int32

def _deg_body(dst_hbm, w_hbm, out_hbm, dstv, wv, stage, deg_sh, sem):
    c = lax.axis_index("c")
    s = lax.axis_index("s")
    wid = c * NS + s

    @pl.when(s == 0)
    def _zero():
        def zrow(i, _):
            stage[pl.ds(i * 16, 16)] = jnp.zeros((16,), F32)
            return 0
        lax.fori_loop(0, N // 16, zrow, 0)
        pltpu.sync_copy(stage, deg_sh)

    plsc.subcore_barrier()

    pltpu.sync_copy(dst_hbm.at[wid], dstv)
    pltpu.sync_copy(w_hbm.at[wid], wv)

    def fire(i, _):
        pltpu.async_copy(wv.at[i], deg_sh.at[dstv.at[i]], sem, add=True)
        return 0
    lax.fori_loop(0, NCHUNK, fire, 0)

    def drain(i, _):
        pltpu.make_async_copy(wv.at[0], deg_sh.at[dstv.at[0]], sem).wait()
        return 0
    lax.fori_loop(0, NCHUNK, drain, 0)

    plsc.subcore_barrier()

    @pl.when(s == 0)
    def _writeout():
        pltpu.sync_copy(deg_sh, stage)
        pltpu.sync_copy(stage, out_hbm.at[c])


def _edge_body(hp_hbm, src_hbm, dst_hbm, w_hbm, out_hbm,
               srcv, dstv, wring, r0, r1, r2, r3, acc_sh,
               g0, g1, g2, g3, s0, s1, s2, s3):
    c = lax.axis_index("c")
    s = lax.axis_index("s")
    wid = c * NS + s
    row0 = s * RPT
    rows = (r0, r1, r2, r3)
    gsem = (g0, g1, g2, g3)
    ssem = (s0, s1, s2, s3)

    def zrow(i, _):
        for j in range(8):
            rows[0][i, pl.ds(j * 16, 16)] = jnp.zeros((16,), F32)
        return 0
    lax.fori_loop(0, WR, zrow, 0)

    def zcopy(q, _):
        pltpu.sync_copy(rows[0].at[pl.ds(0, WR)],
                        acc_sh.at[pl.ds(row0 + q * WR, WR)])
        return 0
    lax.fori_loop(0, RPT // WR, zcopy, 0)

    pltpu.sync_copy(src_hbm.at[wid], srcv)
    pltpu.sync_copy(dst_hbm.at[wid], dstv)

    for b in range(2):
        pltpu.async_copy(hp_hbm.at[srcv.at[b]], rows[b], gsem[b])
        pltpu.async_copy(w_hbm.at[wid, b], wring.at[b], gsem[b])

    plsc.subcore_barrier()

    def quad(p, _):
        for b in range(NBUF):
            ch = NBUF * p + b
            rb, b2 = rows[b], (b + 2) % NBUF
            pltpu.make_async_copy(hp_hbm.at[srcv.at[ch]], rb, gsem[b]).wait()
            pltpu.make_async_copy(
                w_hbm.at[wid, 0], wring.at[b], gsem[b]).wait()

            def scale(e, _, _rb=rb, _b=b):
                wsplat = plsc.load_gather(
                    wring, [jnp.full((16,), _b, I32),
                            jnp.full((16,), 0, I32),
                            jnp.full((16,), e, I32)])
                for j in range(8):
                    sl = pl.ds(j * 16, 16)
                    _rb[e, sl] = _rb[e, sl] * wsplat
                return 0
            lax.fori_loop(0, K, scale, 0, unroll=5)

            pltpu.async_copy(rb, acc_sh.at[dstv.at[ch]], ssem[b], add=True)

            @pl.when(ch >= 2)
            def _(_b2=b2):
                pltpu.make_async_copy(
                    rows[_b2], acc_sh.at[dstv.at[0]], ssem[_b2]).wait()

            @pl.when(ch + 2 < NCHUNK)
            def _(_b2=b2, _ch=ch):
                pltpu.async_copy(
                    hp_hbm.at[srcv.at[_ch + 2]], rows[_b2], gsem[_b2])
                pltpu.async_copy(
                    w_hbm.at[wid, _ch + 2], wring.at[_b2], gsem[_b2])
        return 0
    lax.fori_loop(0, NCHUNK // NBUF, quad, 0)

    for b in ((NCHUNK - 2) % NBUF, (NCHUNK - 1) % NBUF):
        pltpu.make_async_copy(rows[b], acc_sh.at[dstv.at[0]], ssem[b]).wait()

    plsc.subcore_barrier()

    def wcopy(q, _):
        pltpu.sync_copy(acc_sh.at[pl.ds(row0 + q * WR, WR)],
                        rows[0].at[pl.ds(0, WR)])
        pltpu.sync_copy(rows[0].at[pl.ds(0, WR)],
                        out_hbm.at[c, pl.ds(row0 + q * WR, WR)])
        return 0
    lax.fori_loop(0, RPT // WR, wcopy, 0)


@functools.cache
def _sc_kernels():
    mesh = plsc.VectorSubcoreMesh(
        core_axis_name="c", subcore_axis_name="s",
        num_cores=NC, num_subcores=NS)
    deg = pl.kernel(
        _deg_body,
        out_type=jax.ShapeDtypeStruct((NC, N), F32),
        mesh=mesh,
        scratch_types=[
            pltpu.VMEM((NCHUNK, K), I32),
            pltpu.VMEM((NCHUNK, K), F32),
            pltpu.VMEM((N,), F32),
            pltpu.VMEM_SHARED((N,), F32),
            pltpu.SemaphoreType.DMA,
        ],
        compiler_params=pltpu.CompilerParams(needs_layout_passes=False),
    )
    edge = pl.kernel(
        _edge_body,
        out_type=jax.ShapeDtypeStruct((NC, NPAD, H), F32),
        mesh=mesh,
        scratch_types=(
            [
                pltpu.VMEM((NCHUNK, K), I32),
                pltpu.VMEM((NCHUNK, K), I32),
                pltpu.VMEM((NBUF, 1, K), F32),
            ]
            + [pltpu.VMEM((K, H), F32)] * NBUF
            + [pltpu.VMEM_SHARED((NPAD, H), F32)]
            + [pltpu.SemaphoreType.DMA] * (2 * NBUF)
        ),
        compiler_params=pltpu.CompilerParams(
            needs_layout_passes=False, use_tc_tiling_on_sc=False),
    )
    return deg, edge


_RB = 1000
_GRID = N // _RB


def _tc_first_body(degT_ref, x_ref, w1_ref, hp_ref, dinv_ref):
    d = degT_ref[...]
    dv = lax.rsqrt(d[:, 0:1] + d[:, 1:2] + 1.0)
    h = jnp.dot(x_ref[...], w1_ref[...], preferred_element_type=F32)
    hp_ref[...] = h * dv
    dinv_ref[...] = dv


def _tc_first(degT, x, W1):
    return pl.pallas_call(
        _tc_first_body,
        grid=(_GRID,),
        in_specs=[
            pl.BlockSpec((_RB, 2), lambda i: (i, 0)),
            pl.BlockSpec((_RB, D), lambda i: (i, 0)),
            pl.BlockSpec((D, H), lambda i: (0, 0)),
        ],
        out_specs=[
            pl.BlockSpec((_RB, H), lambda i: (i, 0)),
            pl.BlockSpec((_RB, 1), lambda i: (i, 0)),
        ],
        out_shape=[
            jax.ShapeDtypeStruct((NPAD, H), F32),
            jax.ShapeDtypeStruct((N, 1), F32),
        ],
    )(degT, x, W1)


def _tc_mid_body(a0_ref, a1_ref, hp_ref, dinv_ref, b1_ref, w2_ref, out_ref):
    dv = dinv_ref[...]
    z = dv * (a0_ref[...] + a1_ref[...] + hp_ref[...]) + b1_ref[...]
    a = jnp.maximum(z, 0.0)
    out_ref[...] = dv * jnp.dot(a, w2_ref[...], preferred_element_type=F32)


def _tc_mid(a0, a1, hp, dinv, b1, W2):
    return pl.pallas_call(
        _tc_mid_body,
        grid=(_GRID,),
        in_specs=[
            pl.BlockSpec((_RB, H), lambda i: (i, 0)),
            pl.BlockSpec((_RB, H), lambda i: (i, 0)),
            pl.BlockSpec((_RB, H), lambda i: (i, 0)),
            pl.BlockSpec((_RB, 1), lambda i: (i, 0)),
            pl.BlockSpec((1, H), lambda i: (0, 0)),
            pl.BlockSpec((H, H), lambda i: (0, 0)),
        ],
        out_specs=pl.BlockSpec((_RB, H), lambda i: (i, 0)),
        out_shape=jax.ShapeDtypeStruct((NPAD, H), F32),
    )(a0, a1, hp, dinv, b1, W2)


def _tc_last_body(a0_ref, a1_ref, hp_ref, dinv_ref, b2_ref, batch_ref,
                  wout_ref, bout_ref, out_ref, sums, cnt):
    i = pl.program_id(0)
    dv = dinv_ref[...]
    z = dv * (a0_ref[...] + a1_ref[...] + hp_ref[...]) + b2_ref[...]
    a = jnp.maximum(z, 0.0)
    brow = batch_ref[0]
    oh = (lax.broadcasted_iota(I32, (G, _RB), 0) == brow).astype(F32)

    @pl.when(i == 0)
    def _init():
        sums[...] = jnp.zeros((G, H), F32)
        cnt[...] = jnp.zeros((G, 1), F32)

    sums[...] = sums[...] + jnp.dot(oh, a, preferred_element_type=F32)
    cnt[...] = cnt[...] + jnp.sum(oh, axis=1, keepdims=True)

    @pl.when(i == _GRID - 1)
    def _final():
        pooled = sums[...] / jnp.maximum(cnt[...], 1.0)
        out_ref[...] = (jnp.dot(pooled, wout_ref[...],
                                preferred_element_type=F32) + bout_ref[...])


def _tc_last(a0, a1, hp, dinv, b2, batch_row, Wout, bout):
    return pl.pallas_call(
        _tc_last_body,
        grid=(_GRID,),
        in_specs=[
            pl.BlockSpec((_RB, H), lambda i: (i, 0)),
            pl.BlockSpec((_RB, H), lambda i: (i, 0)),
            pl.BlockSpec((_RB, H), lambda i: (i, 0)),
            pl.BlockSpec((_RB, 1), lambda i: (i, 0)),
            pl.BlockSpec((1, H), lambda i: (0, 0)),
            pl.BlockSpec((1, 1, _RB), lambda i: (i, 0, 0)),
            pl.BlockSpec((H, O), lambda i: (0, 0)),
            pl.BlockSpec((1, O), lambda i: (0, 0)),
        ],
        out_specs=pl.BlockSpec((G, O), lambda i: (0, 0)),
        out_shape=jax.ShapeDtypeStruct((G, O), F32),
        scratch_shapes=[pltpu.VMEM((G, H), F32), pltpu.VMEM((G, 1), F32)],
        compiler_params=pltpu.CompilerParams(
            dimension_semantics=("arbitrary",)),
    )(a0, a1, hp, dinv, b2, batch_row, Wout, bout)


def kernel(x, edge_index, edge_weight, batch, W1, b1, W2, b2, Wout, bout):
    deg_kernel, edge_kernel = _sc_kernels()
    src3 = edge_index[0].reshape(NW, NCHUNK, K)
    dst3 = edge_index[1].reshape(NW, NCHUNK, K)
    w3 = edge_weight.reshape(NW, NCHUNK, K)
    w4 = edge_weight.reshape(NW, NCHUNK, 1, K)

    deg_p = deg_kernel(dst3, w3)
    degT = deg_p.T

    hp1, dinv = _tc_first(degT, x, W1)
    acc1 = edge_kernel(hp1, src3, dst3, w4)

    hp2 = _tc_mid(acc1[0], acc1[1], hp1, dinv, b1.reshape(1, H), W2)
    acc2 = edge_kernel(hp2, src3, dst3, w4)

    return _tc_last(acc2[0], acc2[1], hp2, dinv, b2.reshape(1, H),
                    batch.reshape(_GRID, 1, _RB), Wout, bout.reshape(1, O))


# speedup vs baseline: 26.3030x; 1.2556x over previous
import functools

import jax
import jax.numpy as jnp
from jax import lax
from jax.experimental import pallas as pl
from jax.experimental.pallas import tpu as pltpu
from jax.experimental.pallas import tpu_sc as plsc

N, E, D, H, O, G = 10000, 320000, 128, 128, 64, 16
NC, NS = 2, 16
NW = NC * NS
EPW = E // NW
K = 80
NCHUNK = EPW // K
NBUF = 3
NPAD = 10240
RPT = NPAD // NS
WR = 40
F32 = jnp.float32
I32 = jnp.int32

def _deg_body(dst_hbm, w_hbm, out_hbm, dstv, wv, stage, deg_sh, sem):
    c = lax.axis_index("c")
    s = lax.axis_index("s")
    wid = c * NS + s

    @pl.when(s == 0)
    def _zero():
        def zrow(i, _):
            stage[pl.ds(i * 16, 16)] = jnp.zeros((16,), F32)
            return 0
        lax.fori_loop(0, N // 16, zrow, 0)
        pltpu.sync_copy(stage, deg_sh)

    plsc.subcore_barrier()

    pltpu.sync_copy(dst_hbm.at[wid], dstv)
    pltpu.sync_copy(w_hbm.at[wid], wv)

    def fire(i, _):
        pltpu.async_copy(wv.at[i], deg_sh.at[dstv.at[i]], sem, add=True)
        return 0
    lax.fori_loop(0, NCHUNK, fire, 0)

    def drain(i, _):
        pltpu.make_async_copy(wv.at[0], deg_sh.at[dstv.at[0]], sem).wait()
        return 0
    lax.fori_loop(0, NCHUNK, drain, 0)

    plsc.subcore_barrier()

    @pl.when(s == 0)
    def _writeout():
        pltpu.sync_copy(deg_sh, stage)
        pltpu.sync_copy(stage, out_hbm.at[c])


def _edge_body(hp_hbm, src_hbm, dst_hbm, w_hbm, out_hbm,
               srcv, dring, wring, r0, r1, r2, acc_sh,
               g0, g1, g2, s0, s1, s2):
    c = lax.axis_index("c")
    s = lax.axis_index("s")
    wid = c * NS + s
    row0 = s * RPT
    rows = (r0, r1, r2)
    gsem = (g0, g1, g2)
    ssem = (s0, s1, s2)

    def zrow(i, _):
        for j in range(8):
            rows[0][i, pl.ds(j * 16, 16)] = jnp.zeros((16,), F32)
        return 0
    lax.fori_loop(0, WR, zrow, 0)

    def zcopy(q, _):
        pltpu.sync_copy(rows[0].at[pl.ds(0, WR)],
                        acc_sh.at[pl.ds(row0 + q * WR, WR)])
        return 0
    lax.fori_loop(0, RPT // WR, zcopy, 0)

    pltpu.sync_copy(src_hbm.at[wid], srcv)

    for b in range(2):
        pltpu.async_copy(hp_hbm.at[srcv.at[b]], rows[b], gsem[b])
        pltpu.async_copy(dst_hbm.at[wid, b], dring.at[b], gsem[b])
        pltpu.async_copy(w_hbm.at[wid, b], wring.at[b], gsem[b])

    plsc.subcore_barrier()

    def process(ch, b, wait_prev=True, prefetch=True):
        rb, b2 = rows[b], (b + 2) % NBUF
        pltpu.make_async_copy(hp_hbm.at[srcv.at[ch]], rb, gsem[b]).wait()
        pltpu.make_async_copy(dst_hbm.at[wid, 0], dring.at[b], gsem[b]).wait()
        pltpu.make_async_copy(w_hbm.at[wid, 0], wring.at[b], gsem[b]).wait()

        def scale(e, _):
            wsplat = plsc.load_gather(
                wring, [jnp.full((16,), b, I32),
                        jnp.full((16,), e, I32)])
            for j in range(8):
                sl = pl.ds(j * 16, 16)
                rb[e, sl] = rb[e, sl] * wsplat
            return 0
        lax.fori_loop(0, K, scale, 0, unroll=5)

        pltpu.async_copy(rb, acc_sh.at[dring.at[b]], ssem[b], add=True)

        if wait_prev:
            pltpu.make_async_copy(
                rows[b2], acc_sh.at[dring.at[0]], ssem[b2]).wait()
        if prefetch:
            pltpu.async_copy(hp_hbm.at[srcv.at[ch + 2]], rows[b2], gsem[b2])
            pltpu.async_copy(dst_hbm.at[wid, ch + 2], dring.at[b2], gsem[b2])
            pltpu.async_copy(w_hbm.at[wid, ch + 2], wring.at[b2], gsem[b2])

    for b in range(NBUF):
        process(b, b, wait_prev=(b != 0))

    def grp(p, _):
        for b in range(NBUF):
            process(NBUF * p + b, b)
        return 0
    lax.fori_loop(1, NCHUNK // NBUF, grp, 0)
    for r in range(NCHUNK - NCHUNK % NBUF, NCHUNK):
        process(r, r % NBUF, prefetch=(r + 2 < NCHUNK))

    _lb = (NCHUNK - 1) % NBUF
    pltpu.make_async_copy(rows[_lb], acc_sh.at[dring.at[0]], ssem[_lb]).wait()

    plsc.subcore_barrier()

    def wcopy(q, _):
        pltpu.sync_copy(acc_sh.at[pl.ds(row0 + q * WR, WR)],
                        rows[0].at[pl.ds(0, WR)])
        pltpu.sync_copy(rows[0].at[pl.ds(0, WR)],
                        out_hbm.at[c, pl.ds(row0 + q * WR, WR)])
        return 0
    lax.fori_loop(0, RPT // WR, wcopy, 0)


@functools.cache
def _sc_kernels():
    mesh = plsc.VectorSubcoreMesh(
        core_axis_name="c", subcore_axis_name="s",
        num_cores=NC, num_subcores=NS)
    deg = pl.kernel(
        _deg_body,
        out_type=jax.ShapeDtypeStruct((NC, N), F32),
        mesh=mesh,
        scratch_types=[
            pltpu.VMEM((NCHUNK, K), I32),
            pltpu.VMEM((NCHUNK, K), F32),
            pltpu.VMEM((N,), F32),
            pltpu.VMEM_SHARED((N,), F32),
            pltpu.SemaphoreType.DMA,
        ],
        compiler_params=pltpu.CompilerParams(needs_layout_passes=False),
    )
    edge = pl.kernel(
        _edge_body,
        out_type=jax.ShapeDtypeStruct((NC, NPAD, H), F32),
        mesh=mesh,
        scratch_types=(
            [
                pltpu.VMEM((NCHUNK, K), I32),
                pltpu.VMEM((NBUF, K), I32),
                pltpu.VMEM((NBUF, K), F32),
            ]
            + [pltpu.VMEM((K, H), F32)] * NBUF
            + [pltpu.VMEM_SHARED((NPAD, H), F32)]
            + [pltpu.SemaphoreType.DMA] * (2 * NBUF)
        ),
        compiler_params=pltpu.CompilerParams(
            needs_layout_passes=False, use_tc_tiling_on_sc=False),
    )
    return deg, edge


_RB = 1000
_GRID = N // _RB


def _tc_first_body(degT_ref, x_ref, w1_ref, hp_ref, dinv_ref):
    d = degT_ref[...]
    dv = lax.rsqrt(d[:, 0:1] + d[:, 1:2] + 1.0)
    h = jnp.dot(x_ref[...], w1_ref[...], preferred_element_type=F32)
    hp_ref[...] = h * dv
    dinv_ref[...] = dv


def _tc_first(degT, x, W1):
    return pl.pallas_call(
        _tc_first_body,
        grid=(_GRID,),
        in_specs=[
            pl.BlockSpec((_RB, 2), lambda i: (i, 0)),
            pl.BlockSpec((_RB, D), lambda i: (i, 0)),
            pl.BlockSpec((D, H), lambda i: (0, 0)),
        ],
        out_specs=[
            pl.BlockSpec((_RB, H), lambda i: (i, 0)),
            pl.BlockSpec((_RB, 1), lambda i: (i, 0)),
        ],
        out_shape=[
            jax.ShapeDtypeStruct((NPAD, H), F32),
            jax.ShapeDtypeStruct((N, 1), F32),
        ],
    )(degT, x, W1)


def _tc_mid_body(a0_ref, a1_ref, hp_ref, dinv_ref, b1_ref, w2_ref, out_ref):
    dv = dinv_ref[...]
    z = dv * (a0_ref[...] + a1_ref[...] + hp_ref[...]) + b1_ref[...]
    a = jnp.maximum(z, 0.0)
    out_ref[...] = dv * jnp.dot(a, w2_ref[...], preferred_element_type=F32)


def _tc_mid(a0, a1, hp, dinv, b1, W2):
    return pl.pallas_call(
        _tc_mid_body,
        grid=(_GRID,),
        in_specs=[
            pl.BlockSpec((_RB, H), lambda i: (i, 0)),
            pl.BlockSpec((_RB, H), lambda i: (i, 0)),
            pl.BlockSpec((_RB, H), lambda i: (i, 0)),
            pl.BlockSpec((_RB, 1), lambda i: (i, 0)),
            pl.BlockSpec((1, H), lambda i: (0, 0)),
            pl.BlockSpec((H, H), lambda i: (0, 0)),
        ],
        out_specs=pl.BlockSpec((_RB, H), lambda i: (i, 0)),
        out_shape=jax.ShapeDtypeStruct((NPAD, H), F32),
    )(a0, a1, hp, dinv, b1, W2)


def _tc_last_body(a0_ref, a1_ref, hp_ref, dinv_ref, b2_ref, batch_ref,
                  wout_ref, bout_ref, out_ref, sums, cnt):
    i = pl.program_id(0)
    dv = dinv_ref[...]
    z = dv * (a0_ref[...] + a1_ref[...] + hp_ref[...]) + b2_ref[...]
    a = jnp.maximum(z, 0.0)
    brow = batch_ref[0]
    oh = (lax.broadcasted_iota(I32, (G, _RB), 0) == brow).astype(F32)

    @pl.when(i == 0)
    def _init():
        sums[...] = jnp.zeros((G, H), F32)
        cnt[...] = jnp.zeros((G, 1), F32)

    sums[...] = sums[...] + jnp.dot(oh, a, preferred_element_type=F32)
    cnt[...] = cnt[...] + jnp.sum(oh, axis=1, keepdims=True)

    @pl.when(i == _GRID - 1)
    def _final():
        pooled = sums[...] / jnp.maximum(cnt[...], 1.0)
        out_ref[...] = (jnp.dot(pooled, wout_ref[...],
                                preferred_element_type=F32) + bout_ref[...])


def _tc_last(a0, a1, hp, dinv, b2, batch_row, Wout, bout):
    return pl.pallas_call(
        _tc_last_body,
        grid=(_GRID,),
        in_specs=[
            pl.BlockSpec((_RB, H), lambda i: (i, 0)),
            pl.BlockSpec((_RB, H), lambda i: (i, 0)),
            pl.BlockSpec((_RB, H), lambda i: (i, 0)),
            pl.BlockSpec((_RB, 1), lambda i: (i, 0)),
            pl.BlockSpec((1, H), lambda i: (0, 0)),
            pl.BlockSpec((1, 1, _RB), lambda i: (i, 0, 0)),
            pl.BlockSpec((H, O), lambda i: (0, 0)),
            pl.BlockSpec((1, O), lambda i: (0, 0)),
        ],
        out_specs=pl.BlockSpec((G, O), lambda i: (0, 0)),
        out_shape=jax.ShapeDtypeStruct((G, O), F32),
        scratch_shapes=[pltpu.VMEM((G, H), F32), pltpu.VMEM((G, 1), F32)],
        compiler_params=pltpu.CompilerParams(
            dimension_semantics=("arbitrary",)),
    )(a0, a1, hp, dinv, b2, batch_row, Wout, bout)


def kernel(x, edge_index, edge_weight, batch, W1, b1, W2, b2, Wout, bout):
    deg_kernel, edge_kernel = _sc_kernels()
    src3 = edge_index[0].reshape(NW, NCHUNK, K)
    dst3 = edge_index[1].reshape(NW, NCHUNK, K)
    w3 = edge_weight.reshape(NW, NCHUNK, K)

    deg_p = deg_kernel(dst3, w3)
    degT = deg_p.T

    hp1, dinv = _tc_first(degT, x, W1)
    acc1 = edge_kernel(hp1, src3, dst3, w3)

    hp2 = _tc_mid(acc1[0], acc1[1], hp1, dinv, b1.reshape(1, H), W2)
    acc2 = edge_kernel(hp2, src3, dst3, w3)

    return _tc_last(acc2[0], acc2[1], hp2, dinv, b2.reshape(1, H),
                    batch.reshape(_GRID, 1, _RB), Wout, bout.reshape(1, O))
